# Initial kernel scaffold; baseline (speedup 1.0000x reference)
import jax
import jax.numpy as jnp
from jax.experimental import pallas as pl


def kernel(x, params, edge_index, batch, target):
    raise NotImplementedError("write your pallas kernel here")


# speedup vs baseline: 1.5689x; 1.5689x over previous
import functools
from functools import partial

import jax
import jax.numpy as jnp
from jax import lax
from jax.experimental import pallas as pl
from jax.experimental.pallas import tpu as pltpu

_CH = 512


def _attn_body(starts_ref, counts_ref, vf_ref, c_ref, hp_ref, o_ref, xs_ref,
               xbuf, sem, *, ch, with_pool):
    b = pl.program_id(0)
    start = starts_ref[b]
    cnt = counts_ref[b]
    vf = vf_ref[0]
    cb = c_ref[0]
    nq = cb.shape[0]
    d = cb.shape[1]
    scale = 1.0 / (float(d) ** 0.5)

    m0 = jnp.full((nq, 1), -jnp.inf, jnp.float32)
    z0 = jnp.zeros((nq, 1), jnp.float32)
    a0 = jnp.zeros((nq, d), jnp.float32)
    x0 = jnp.zeros((1, d), jnp.float32)

    nt = (cnt + ch - 1) // ch

    def body(i, carry):
        m, z, acc, xsum = carry
        cp = pltpu.make_async_copy(hp_ref.at[pl.ds(start + i * ch, ch)],
                                   xbuf, sem)
        cp.start()
        cp.wait()
        x = xbuf[...]
        rem = cnt - i * ch
        s = lax.dot_general(cb, x, (((1,), (1,)), ((), ())),
                            precision=lax.Precision.HIGHEST) * scale
        col = lax.broadcasted_iota(jnp.int32, (nq, ch), 1)
        valid = col < rem
        s = jnp.where(valid, s, -jnp.inf)
        m_new = jnp.maximum(m, jnp.max(s, axis=1, keepdims=True))
        alpha = jnp.exp(m - m_new)
        p = jnp.where(valid, jnp.exp(s - m_new), 0.0)
        z = z * alpha + jnp.sum(p, axis=1, keepdims=True)
        acc = acc * alpha + lax.dot_general(p, x, (((1,), (0,)), ((), ())),
                                           precision=lax.Precision.HIGHEST)
        if with_pool:
            rcol = lax.broadcasted_iota(jnp.int32, (1, ch), 1)
            w = jnp.where(rcol < rem, 1.0, 0.0)
            xsum = xsum + lax.dot_general(w, x, (((1,), (0,)), ((), ())),
                                          precision=lax.Precision.HIGHEST)
        return m_new, z, acc, xsum

    m, z, acc, xsum = lax.fori_loop(0, nt, body, (m0, z0, a0, x0))

    pad_m = jnp.where(cnt < vf, 0.0, -jnp.inf)
    m_f = jnp.maximum(m, pad_m)
    r = jnp.exp(m - m_f)
    z = z * r + (vf - cnt).astype(jnp.float32) * jnp.exp(-m_f)
    acc = acc * r
    o_ref[0] = acc / z
    if with_pool:
        xs_ref[0] = xsum


def _ragged_attn(c, hp, starts, counts, vf, *, with_pool):
    b, nq, d = c.shape
    kern = partial(_attn_body, ch=_CH, with_pool=with_pool)
    out_shape = [jax.ShapeDtypeStruct((b, nq, d), jnp.float32),
                 jax.ShapeDtypeStruct((b, 1, d), jnp.float32)]
    outs = pl.pallas_call(
        kern,
        grid=(b,),
        in_specs=[
            pl.BlockSpec(memory_space=pltpu.SMEM),
            pl.BlockSpec(memory_space=pltpu.SMEM),
            pl.BlockSpec(memory_space=pltpu.SMEM),
            pl.BlockSpec((1, nq, d), lambda i: (i, 0, 0)),
            pl.BlockSpec(memory_space=pl.ANY),
        ],
        out_specs=[
            pl.BlockSpec((1, nq, d), lambda i: (i, 0, 0)),
            pl.BlockSpec((1, 1, d), lambda i: (i, 0, 0)),
        ],
        out_shape=out_shape,
        scratch_shapes=[
            pltpu.VMEM((_CH, d), jnp.float32),
            pltpu.SemaphoreType.DMA,
        ],
    )(starts, counts, vf, c, hp)
    if with_pool:
        return outs[0], outs[1].reshape(b, d)
    return outs[0]


def kernel(x, params, edge_index, batch, target):
    p = params
    n_nodes, d_in = x.shape
    bsz = target.shape[0]
    src, dst = edge_index[0], edge_index[1]

    counts = jnp.bincount(batch, length=bsz).astype(jnp.int32)
    v_freq = jnp.max(counts).reshape(1)
    starts = jnp.searchsorted(batch, jnp.arange(bsz)).astype(jnp.int32)

    def agg(h):
        return jax.ops.segment_sum(h[src], dst, num_segments=n_nodes)

    def gin(h, i):
        z = h + agg(h)
        z = jax.nn.relu(z @ p['gin%d_W1' % i] + p['gin%d_b1' % i])
        return z @ p['gin%d_W2' % i] + p['gin%d_b2' % i]

    def bn(h, i):
        m = h.mean(0)
        v = h.var(0)
        return (h - m) / jnp.sqrt(v + 1e-5) * p['bn%d_g' % i] + p['bn%d_b' % i]

    h = bn(jax.nn.relu(gin(x, 1)), 1)
    h = bn(jax.nn.relu(gin(h, 2)), 2)
    h = bn(jax.nn.relu(gin(h, 3)), 3)

    emb = p['emb'][target]
    c1 = lax.conv_general_dilated(
        emb, p['cxt1_w'], (1,), 'VALID',
        dimension_numbers=('NCH', 'OIH', 'NCH')) + p['cxt1_b'][None, :, None]

    dim = h.shape[1]
    pad = jnp.zeros((_CH, dim), jnp.float32)
    hp = jnp.concatenate([h, pad], axis=0)
    o1 = _ragged_attn(c1, hp, starts, counts, v_freq, with_pool=False)

    h = bn(jax.nn.relu(gin(h, 4)), 4)
    h = bn(jax.nn.relu(gin(h, 5)), 5)

    c2 = lax.conv_general_dilated(
        o1, p['cxt2_w'], (1,), [(3, 4)],
        dimension_numbers=('NCH', 'OIH', 'NCH')) + p['cxt2_b'][None, :, None]

    hp2 = jnp.concatenate([h, pad], axis=0)
    o2, xg = _ragged_attn(c2, hp2, starts, counts, v_freq, with_pool=True)

    xg = jax.nn.relu(xg @ p['fc1_xd_w'] + p['fc1_xd_b'])
    nf = o2.shape[1]
    xt = o2.reshape(bsz, nf * dim) @ p['fc1_xt_w'] + p['fc1_xt_b']
    xc = jnp.concatenate([xg, xt], axis=1)
    xc = jax.nn.relu(xc @ p['fc1_w'] + p['fc1_b'])
    xc = jax.nn.relu(xc @ p['fc2_w'] + p['fc2_b'])
    return xc @ p['out_w'] + p['out_b']
